# baseline (speedup 1.0000x reference)
import functools

import jax
import jax.numpy as jnp
from jax import lax
from jax.experimental import pallas as pl
from jax.experimental.pallas import tpu as pltpu
from jax.experimental.pallas import tpu_sc as plsc

VOCAB = 100000
EMB = 16
BATCH = 1024
CTX = 20

_NC = 2
_NS = 16
_NW = _NC * _NS
_BPW = BATCH // _NW
_IPW = _BPW * CTX
_GCHUNK = 128
_NCHUNK = _IPW // _GCHUNK


def _make_mean_kernel():
    mesh = plsc.VectorSubcoreMesh(core_axis_name="c", subcore_axis_name="s")

    @functools.partial(
        pl.kernel,
        mesh=mesh,
        out_type=jax.ShapeDtypeStruct((BATCH, EMB), jnp.float32),
        scratch_types=[
            pltpu.VMEM((_IPW,), jnp.int32),
            pltpu.VMEM((_IPW, EMB), jnp.float32),
            pltpu.VMEM((_BPW, EMB), jnp.float32),
            pltpu.SemaphoreType.DMA,
        ],
        compiler_params=pltpu.CompilerParams(use_tc_tiling_on_sc=False),
    )
    def mean_kernel(idx_hbm, table_hbm, out_hbm, idx_v, rows_v, mean_v, sem):
        wid = lax.axis_index("s") * _NC + lax.axis_index("c")
        pltpu.sync_copy(idx_hbm.at[pl.ds(wid * _IPW, _IPW)], idx_v)
        copies = []
        for c in range(_NCHUNK):
            copies.append(
                pltpu.async_copy(
                    table_hbm.at[idx_v.at[pl.ds(c * _GCHUNK, _GCHUNK)]],
                    rows_v.at[pl.ds(c * _GCHUNK, _GCHUNK)],
                    sem,
                )
            )
        for cp in copies:
            cp.wait()

        scale = jnp.float32(1.0 / CTX)

        def body(i, carry):
            acc = rows_v[i * CTX, :]
            for j in range(1, CTX):
                acc = acc + rows_v[i * CTX + j, :]
            mean_v[i, :] = acc * scale
            return carry

        lax.fori_loop(0, _BPW, body, 0)
        pltpu.sync_copy(mean_v, out_hbm.at[pl.ds(wid * _BPW, _BPW)])

    return mean_kernel


_mean_kernel = _make_mean_kernel()

_VBLK = 2048
_NVB = (VOCAB + _VBLK - 1) // _VBLK


def _proj_body(mean_ref, w_ref, b_ref, out_ref):
    out_ref[...] = (
        lax.dot_general(
            mean_ref[...],
            w_ref[...],
            dimension_numbers=(((1,), (1,)), ((), ())),
            preferred_element_type=jnp.float32,
        )
        + b_ref[...]
    )


def _projection(mean, w, b2d):
    return pl.pallas_call(
        _proj_body,
        grid=(_NVB,),
        in_specs=[
            pl.BlockSpec((BATCH, EMB), lambda i: (0, 0)),
            pl.BlockSpec((_VBLK, EMB), lambda i: (i, 0)),
            pl.BlockSpec((1, _VBLK), lambda i: (0, i)),
        ],
        out_specs=pl.BlockSpec((BATCH, _VBLK), lambda i: (0, i)),
        out_shape=jax.ShapeDtypeStruct((BATCH, VOCAB), jnp.float32),
        compiler_params=pltpu.CompilerParams(
            dimension_semantics=("arbitrary",),
        ),
    )(mean, w, b2d)


def kernel(context_words, embeddings, linear_w, linear_b):
    idx = context_words.astype(jnp.int32).reshape(-1)
    mean = _mean_kernel(idx, embeddings)
    b2d = linear_b.reshape(1, VOCAB)
    return _projection(mean, linear_w, b2d)


# speedup vs baseline: 1.0045x; 1.0045x over previous
import functools

import jax
import jax.numpy as jnp
from jax import lax
from jax.experimental import pallas as pl
from jax.experimental.pallas import tpu as pltpu
from jax.experimental.pallas import tpu_sc as plsc

VOCAB = 100000
EMB = 16
BATCH = 1024
CTX = 20

_NC = 2
_NS = 16
_NW = _NC * _NS
_BPW = BATCH // _NW
_IPW = _BPW * CTX
_GCHUNK = 128
_NCHUNK = _IPW // _GCHUNK


def _make_mean_kernel():
    mesh = plsc.VectorSubcoreMesh(core_axis_name="c", subcore_axis_name="s")

    @functools.partial(
        pl.kernel,
        mesh=mesh,
        out_type=jax.ShapeDtypeStruct((BATCH, EMB), jnp.float32),
        scratch_types=[
            pltpu.VMEM((_IPW,), jnp.int32),
            pltpu.VMEM((_IPW, EMB), jnp.float32),
            pltpu.VMEM((_BPW, EMB), jnp.float32),
            pltpu.SemaphoreType.DMA,
        ],
        compiler_params=pltpu.CompilerParams(use_tc_tiling_on_sc=False),
    )
    def mean_kernel(idx_hbm, table_hbm, out_hbm, idx_v, rows_v, mean_v, sem):
        wid = lax.axis_index("s") * _NC + lax.axis_index("c")
        pltpu.sync_copy(idx_hbm.at[pl.ds(wid * _IPW, _IPW)], idx_v)
        copies = []
        for c in range(_NCHUNK):
            copies.append(
                pltpu.async_copy(
                    table_hbm.at[idx_v.at[pl.ds(c * _GCHUNK, _GCHUNK)]],
                    rows_v.at[pl.ds(c * _GCHUNK, _GCHUNK)],
                    sem,
                )
            )
        for cp in copies:
            cp.wait()

        scale = jnp.float32(1.0 / CTX)

        def body(i, carry):
            acc = rows_v[i * CTX, :]
            for j in range(1, CTX):
                acc = acc + rows_v[i * CTX + j, :]
            mean_v[i, :] = acc * scale
            return carry

        lax.fori_loop(0, _BPW, body, 0)
        pltpu.sync_copy(mean_v, out_hbm.at[pl.ds(wid * _BPW, _BPW)])

    return mean_kernel


_mean_kernel = _make_mean_kernel()

_VBLK = 4096
_NVB = (VOCAB + _VBLK - 1) // _VBLK


def _proj_body(mean_ref, w_ref, b_ref, out_ref):
    out_ref[...] = (
        lax.dot_general(
            mean_ref[...],
            w_ref[...],
            dimension_numbers=(((1,), (1,)), ((), ())),
            preferred_element_type=jnp.float32,
        )
        + b_ref[...]
    )


def _projection(mean, w, b2d):
    return pl.pallas_call(
        _proj_body,
        grid=(_NVB,),
        in_specs=[
            pl.BlockSpec((BATCH, EMB), lambda i: (0, 0)),
            pl.BlockSpec((_VBLK, EMB), lambda i: (i, 0)),
            pl.BlockSpec((1, _VBLK), lambda i: (0, i)),
        ],
        out_specs=pl.BlockSpec((BATCH, _VBLK), lambda i: (0, i)),
        out_shape=jax.ShapeDtypeStruct((BATCH, VOCAB), jnp.float32),
        compiler_params=pltpu.CompilerParams(
            dimension_semantics=("arbitrary",),
        ),
    )(mean, w, b2d)


def kernel(context_words, embeddings, linear_w, linear_b):
    idx = context_words.astype(jnp.int32).reshape(-1)
    mean = _mean_kernel(idx, embeddings)
    b2d = linear_b.reshape(1, VOCAB)
    return _projection(mean, linear_w, b2d)


# speedup vs baseline: 1.0151x; 1.0106x over previous
import functools

import jax
import jax.numpy as jnp
from jax import lax
from jax.experimental import pallas as pl
from jax.experimental.pallas import tpu as pltpu
from jax.experimental.pallas import tpu_sc as plsc

VOCAB = 100000
EMB = 16
BATCH = 1024
CTX = 20

_NC = 2
_NS = 16
_NW = _NC * _NS
_BPW = BATCH // _NW
_IPW = _BPW * CTX
_GCHUNK = 128
_NCHUNK = _IPW // _GCHUNK


def _make_mean_kernel():
    mesh = plsc.VectorSubcoreMesh(core_axis_name="c", subcore_axis_name="s")

    @functools.partial(
        pl.kernel,
        mesh=mesh,
        out_type=jax.ShapeDtypeStruct((BATCH, EMB), jnp.float32),
        scratch_types=[
            pltpu.VMEM((_IPW,), jnp.int32),
            pltpu.VMEM((_IPW, EMB), jnp.float32),
            pltpu.VMEM((_BPW, EMB), jnp.float32),
            pltpu.SemaphoreType.DMA,
        ],
        compiler_params=pltpu.CompilerParams(use_tc_tiling_on_sc=False),
    )
    def mean_kernel(idx_hbm, table_hbm, out_hbm, idx_v, rows_v, mean_v, sem):
        wid = lax.axis_index("s") * _NC + lax.axis_index("c")
        pltpu.sync_copy(idx_hbm.at[pl.ds(wid * _IPW, _IPW)], idx_v)
        copies = []
        for c in range(_NCHUNK):
            copies.append(
                pltpu.async_copy(
                    table_hbm.at[idx_v.at[pl.ds(c * _GCHUNK, _GCHUNK)]],
                    rows_v.at[pl.ds(c * _GCHUNK, _GCHUNK)],
                    sem,
                )
            )
        for cp in copies:
            cp.wait()

        scale = jnp.float32(1.0 / CTX)

        def body(i, carry):
            acc = rows_v[i * CTX, :]
            for j in range(1, CTX):
                acc = acc + rows_v[i * CTX + j, :]
            mean_v[i, :] = acc * scale
            return carry

        lax.fori_loop(0, _BPW, body, 0)
        pltpu.sync_copy(mean_v, out_hbm.at[pl.ds(wid * _BPW, _BPW)])

    return mean_kernel


_mean_kernel = _make_mean_kernel()

_VBLK = 2048
_NVB = (VOCAB + _VBLK - 1) // _VBLK


def _proj_body(mean_ref, w_ref, b_ref, out_ref):
    out_ref[...] = (
        lax.dot_general(
            mean_ref[...],
            w_ref[...],
            dimension_numbers=(((1,), (1,)), ((), ())),
            preferred_element_type=jnp.float32,
        )
        + b_ref[...]
    )


def _projection(mean, w, b2d):
    return pl.pallas_call(
        _proj_body,
        grid=(_NVB,),
        in_specs=[
            pl.BlockSpec((BATCH, EMB), lambda i: (0, 0)),
            pl.BlockSpec((_VBLK, EMB), lambda i: (i, 0)),
            pl.BlockSpec((1, _VBLK), lambda i: (0, i)),
        ],
        out_specs=pl.BlockSpec((BATCH, _VBLK), lambda i: (0, i)),
        out_shape=jax.ShapeDtypeStruct((BATCH, VOCAB), jnp.float32),
        compiler_params=pltpu.CompilerParams(
            dimension_semantics=("arbitrary",),
        ),
    )(mean, w, b2d)


def kernel(context_words, embeddings, linear_w, linear_b):
    idx = context_words.astype(jnp.int32).reshape(-1)
    mean = _mean_kernel(idx, embeddings)
    b2d = linear_b.reshape(1, VOCAB)
    return _projection(mean, linear_w, b2d)


# speedup vs baseline: 2.9466x; 2.9027x over previous
import functools

import jax
import jax.numpy as jnp
from jax import lax
from jax.experimental import pallas as pl
from jax.experimental.pallas import tpu as pltpu
from jax.experimental.pallas import tpu_sc as plsc

VOCAB = 100000
EMB = 16
BATCH = 1024
CTX = 20

_NC = 2
_NS = 16
_NW = _NC * _NS
_BPW = BATCH // _NW
_IPW = _BPW * CTX
_GCHUNK = 128
_NCHUNK = _IPW // _GCHUNK


def _make_mean_kernel():
    mesh = plsc.VectorSubcoreMesh(core_axis_name="c", subcore_axis_name="s")

    @functools.partial(
        pl.kernel,
        mesh=mesh,
        out_type=jax.ShapeDtypeStruct((BATCH, EMB), jnp.float32),
        scratch_types=[
            pltpu.VMEM((_IPW,), jnp.int32),
            pltpu.VMEM((_IPW, EMB), jnp.float32),
            pltpu.VMEM((_BPW, EMB), jnp.float32),
            pltpu.SemaphoreType.DMA,
        ],
        compiler_params=pltpu.CompilerParams(use_tc_tiling_on_sc=False),
    )
    def mean_kernel(idx_hbm, table_hbm, out_hbm, idx_v, rows_v, mean_v, sem):
        wid = lax.axis_index("s") * _NC + lax.axis_index("c")
        pltpu.sync_copy(idx_hbm.at[pl.ds(wid * _IPW, _IPW)], idx_v)
        copies = []
        for c in range(_NCHUNK):
            copies.append(
                pltpu.async_copy(
                    table_hbm.at[idx_v.at[pl.ds(c * _GCHUNK, _GCHUNK)]],
                    rows_v.at[pl.ds(c * _GCHUNK, _GCHUNK)],
                    sem,
                )
            )
        for cp in copies:
            cp.wait()

        scale = jnp.float32(1.0 / CTX)

        def body(i, carry):
            acc = rows_v[i * CTX, :]
            for j in range(1, CTX):
                acc = acc + rows_v[i * CTX + j, :]
            mean_v[i, :] = acc * scale
            return carry

        lax.fori_loop(0, _BPW, body, 0)
        pltpu.sync_copy(mean_v, out_hbm.at[pl.ds(wid * _BPW, _BPW)])

    return mean_kernel


_mean_kernel = _make_mean_kernel()

_VBLK = 2048
_NVB = (VOCAB + _VBLK - 1) // _VBLK


_KA = EMB + 1


def _proj_body(mean_ref, wt_ref, out_ref):
    out_ref[...] = lax.dot_general(
        wt_ref[...],
        mean_ref[...],
        dimension_numbers=(((0,), (1,)), ((), ())),
        preferred_element_type=jnp.float32,
    )


def _projection_t(mean_aug, wt_aug):
    return pl.pallas_call(
        _proj_body,
        grid=(_NVB,),
        in_specs=[
            pl.BlockSpec((BATCH, _KA), lambda i: (0, 0)),
            pl.BlockSpec((_KA, _VBLK), lambda i: (0, i)),
        ],
        out_specs=pl.BlockSpec((_VBLK, BATCH), lambda i: (i, 0)),
        out_shape=jax.ShapeDtypeStruct((VOCAB, BATCH), jnp.float32),
        compiler_params=pltpu.CompilerParams(
            dimension_semantics=("arbitrary",),
        ),
    )(mean_aug, wt_aug)


def kernel(context_words, embeddings, linear_w, linear_b):
    idx = context_words.astype(jnp.int32).reshape(-1)
    mean = _mean_kernel(idx, embeddings)
    wt_aug = jnp.concatenate([linear_w.T, linear_b.reshape(1, VOCAB)], axis=0)
    mean_aug = jnp.concatenate(
        [mean, jnp.ones((BATCH, 1), jnp.float32)], axis=1
    )
    out_t = _projection_t(mean_aug, wt_aug)
    return out_t.T


# speedup vs baseline: 2.9624x; 1.0054x over previous
import functools

import jax
import jax.numpy as jnp
from jax import lax
from jax.experimental import pallas as pl
from jax.experimental.pallas import tpu as pltpu
from jax.experimental.pallas import tpu_sc as plsc

VOCAB = 100000
EMB = 16
BATCH = 1024
CTX = 20

_NC = 2
_NS = 16
_NW = _NC * _NS
_BPW = BATCH // _NW
_WPW = _BPW * CTX * EMB
_GCHUNK = 128
_NCHUNK = _WPW // _GCHUNK
_GUNROLL = 8


def _make_mean_kernel():
    mesh = plsc.VectorSubcoreMesh(core_axis_name="c", subcore_axis_name="s")

    @functools.partial(
        pl.kernel,
        mesh=mesh,
        out_type=jax.ShapeDtypeStruct((BATCH, EMB), jnp.float32),
        scratch_types=[
            pltpu.VMEM((_WPW,), jnp.int32),
            pltpu.VMEM((_WPW,), jnp.float32),
            pltpu.VMEM((_BPW, EMB), jnp.float32),
            pltpu.SemaphoreType.DMA,
        ],
        compiler_params=pltpu.CompilerParams(use_tc_tiling_on_sc=False),
    )
    def mean_kernel(widx_hbm, table_hbm, out_hbm, idx_v, vals_v, mean_v, sem):
        wid = lax.axis_index("s") * _NC + lax.axis_index("c")
        pltpu.sync_copy(widx_hbm.at[pl.ds(wid * _WPW, _WPW)], idx_v)

        def fire(step, carry):
            base = step * _GUNROLL * _GCHUNK
            copies = []
            for u in range(_GUNROLL):
                off = base + u * _GCHUNK
                copies.append(
                    pltpu.async_copy(
                        table_hbm.at[idx_v.at[pl.ds(off, _GCHUNK)]],
                        vals_v.at[pl.ds(off, _GCHUNK)],
                        sem,
                    )
                )
            for cp in copies:
                cp.wait()
            return carry

        lax.fori_loop(0, _NCHUNK // _GUNROLL, fire, 0)

        scale = jnp.float32(1.0 / CTX)

        def body(i, carry):
            acc = vals_v[pl.ds(i * (CTX * EMB), EMB)]
            for j in range(1, CTX):
                acc = acc + vals_v[pl.ds(i * (CTX * EMB) + j * EMB, EMB)]
            mean_v[i, :] = acc * scale
            return carry

        lax.fori_loop(0, _BPW, body, 0)
        pltpu.sync_copy(mean_v, out_hbm.at[pl.ds(wid * _BPW, _BPW)])

    return mean_kernel


_mean_kernel = _make_mean_kernel()

_VBLK = 2048
_NVB = (VOCAB + _VBLK - 1) // _VBLK
_KA = EMB + 1


def _proj_body(mean_ref, wt_ref, out_ref):
    out_ref[...] = lax.dot_general(
        wt_ref[...],
        mean_ref[...],
        dimension_numbers=(((0,), (1,)), ((), ())),
        preferred_element_type=jnp.float32,
    )


def _projection_t(mean_aug, wt_aug):
    return pl.pallas_call(
        _proj_body,
        grid=(_NVB,),
        in_specs=[
            pl.BlockSpec((BATCH, _KA), lambda i: (0, 0)),
            pl.BlockSpec((_KA, _VBLK), lambda i: (0, i)),
        ],
        out_specs=pl.BlockSpec((_VBLK, BATCH), lambda i: (i, 0)),
        out_shape=jax.ShapeDtypeStruct((VOCAB, BATCH), jnp.float32),
        compiler_params=pltpu.CompilerParams(
            dimension_semantics=("arbitrary",),
        ),
    )(mean_aug, wt_aug)


def kernel(context_words, embeddings, linear_w, linear_b):
    idx = context_words.astype(jnp.int32).reshape(-1)
    widx = (
        idx[:, None] + (jnp.arange(EMB, dtype=jnp.int32) * VOCAB)[None, :]
    ).reshape(-1)
    table_lin = embeddings.T.reshape(-1)
    mean = _mean_kernel(widx, table_lin)
    wt_aug = jnp.concatenate([linear_w.T, linear_b.reshape(1, VOCAB)], axis=0)
    mean_aug = jnp.concatenate(
        [mean, jnp.ones((BATCH, 1), jnp.float32)], axis=1
    )
    out_t = _projection_t(mean_aug, wt_aug)
    return out_t.T


# speedup vs baseline: 3.0517x; 1.0301x over previous
import functools

import jax
import jax.numpy as jnp
from jax import lax
from jax.experimental import pallas as pl
from jax.experimental.pallas import tpu as pltpu
from jax.experimental.pallas import tpu_sc as plsc

VOCAB = 100000
EMB = 16
BATCH = 1024
CTX = 20

_NC = 2
_NS = 16
_NW = _NC * _NS
_BPW = BATCH // _NW
_WPW = _BPW * CTX * EMB
_GCHUNK = 128
_NCHUNK = _WPW // _GCHUNK
_GUNROLL = 8


def _make_mean_kernel():
    mesh = plsc.VectorSubcoreMesh(core_axis_name="c", subcore_axis_name="s")

    @functools.partial(
        pl.kernel,
        mesh=mesh,
        out_type=jax.ShapeDtypeStruct((BATCH, EMB), jnp.float32),
        scratch_types=[
            pltpu.VMEM((_WPW,), jnp.int32),
            pltpu.VMEM((_WPW,), jnp.float32),
            pltpu.VMEM((_BPW, EMB), jnp.float32),
            pltpu.SemaphoreType.DMA,
            pltpu.SemaphoreType.DMA,
        ],
        compiler_params=pltpu.CompilerParams(use_tc_tiling_on_sc=False),
    )
    def mean_kernel(
        widx_hbm, table_hbm, out_hbm, idx_v, vals_v, mean_v, sem, stage_sem
    ):
        wid = lax.axis_index("s") * _NC + lax.axis_index("c")
        half = _WPW // 2
        stage0 = pltpu.async_copy(
            widx_hbm.at[pl.ds(wid * _WPW, half)],
            idx_v.at[pl.ds(0, half)],
            stage_sem,
        )
        stage1 = pltpu.async_copy(
            widx_hbm.at[pl.ds(wid * _WPW + half, half)],
            idx_v.at[pl.ds(half, half)],
            stage_sem,
        )

        def fire(step, carry):
            base = step * _GUNROLL * _GCHUNK
            for u in range(_GUNROLL):
                off = base + u * _GCHUNK
                pltpu.async_copy(
                    table_hbm.at[idx_v.at[pl.ds(off, _GCHUNK)]],
                    vals_v.at[pl.ds(off, _GCHUNK)],
                    sem,
                )
            return carry

        nhalf_steps = _NCHUNK // _GUNROLL // 2
        stage0.wait()
        lax.fori_loop(0, nhalf_steps, fire, 0)
        stage1.wait()
        lax.fori_loop(nhalf_steps, 2 * nhalf_steps, fire, 0)
        pltpu.make_async_copy(table_hbm.at[pl.ds(0, _WPW)], vals_v, sem).wait()

        scale = jnp.float32(1.0 / CTX)

        def body(i, carry):
            acc = vals_v[pl.ds(i * (CTX * EMB), EMB)]
            for j in range(1, CTX):
                acc = acc + vals_v[pl.ds(i * (CTX * EMB) + j * EMB, EMB)]
            mean_v[i, :] = acc * scale
            return carry

        lax.fori_loop(0, _BPW, body, 0)
        pltpu.sync_copy(mean_v, out_hbm.at[pl.ds(wid * _BPW, _BPW)])

    return mean_kernel


_mean_kernel = _make_mean_kernel()

_VBLK = 2048
_NVB = (VOCAB + _VBLK - 1) // _VBLK
_KA = EMB + 1


def _proj_body(mean_ref, wt_ref, out_ref):
    out_ref[...] = lax.dot_general(
        wt_ref[...],
        mean_ref[...],
        dimension_numbers=(((0,), (1,)), ((), ())),
        preferred_element_type=jnp.float32,
    )


def _projection_t(mean_aug, wt_aug):
    return pl.pallas_call(
        _proj_body,
        grid=(_NVB,),
        in_specs=[
            pl.BlockSpec((BATCH, _KA), lambda i: (0, 0)),
            pl.BlockSpec((_KA, _VBLK), lambda i: (0, i)),
        ],
        out_specs=pl.BlockSpec((_VBLK, BATCH), lambda i: (i, 0)),
        out_shape=jax.ShapeDtypeStruct((VOCAB, BATCH), jnp.float32),
        compiler_params=pltpu.CompilerParams(
            dimension_semantics=("arbitrary",),
        ),
    )(mean_aug, wt_aug)


def kernel(context_words, embeddings, linear_w, linear_b):
    idx = context_words.astype(jnp.int32).reshape(-1)
    widx = (
        idx[:, None] + (jnp.arange(EMB, dtype=jnp.int32) * VOCAB)[None, :]
    ).reshape(-1)
    table_lin = embeddings.T.reshape(-1)
    mean = _mean_kernel(widx, table_lin)
    wt_aug = jnp.concatenate([linear_w.T, linear_b.reshape(1, VOCAB)], axis=0)
    mean_aug = jnp.concatenate(
        [mean, jnp.ones((BATCH, 1), jnp.float32)], axis=1
    )
    out_t = _projection_t(mean_aug, wt_aug)
    return out_t.T
